# baseline (speedup 1.0000x reference)
import functools

import jax
import jax.numpy as jnp
from jax import lax
from jax.experimental import pallas as pl
from jax.experimental.pallas import tpu as pltpu
from jax.experimental.pallas import tpu_sc as plsc

N_NODES_C = 10000
E_C = 320000
EMB_C = 128
RBF_PAD = 8


_EDGE_BLK = 8000


def _edge_kernel(rbf_ref, m_ref, wrbf_ref, tmp_ref):
    p = jnp.dot(rbf_ref[...], wrbf_ref[...],
                preferred_element_type=jnp.float32)
    tmp_ref[...] = m_ref[...] * p


def _edge_stage(rbf_p, m, wrbf_t):
    e = m.shape[0]
    return pl.pallas_call(
        _edge_kernel,
        grid=(e // _EDGE_BLK,),
        in_specs=[
            pl.BlockSpec((_EDGE_BLK, RBF_PAD), lambda i: (i, 0)),
            pl.BlockSpec((_EDGE_BLK, EMB_C), lambda i: (i, 0)),
            pl.BlockSpec((RBF_PAD, EMB_C), lambda i: (0, 0)),
        ],
        out_specs=pl.BlockSpec((_EDGE_BLK, EMB_C), lambda i: (i, 0)),
        out_shape=jax.ShapeDtypeStruct((e, EMB_C), jnp.float32),
    )(rbf_p, m, wrbf_t)


_CHUNK = 80
_N_CORES = 2
_N_SUB = 16
_E_PER_SUB = E_C // (_N_CORES * _N_SUB)
_N_CHUNKS = _E_PER_SUB // _CHUNK
_N_PAD = 10240
_ROWS_PER_SUB = _N_PAD // _N_SUB


def _make_seg_kernel():
    mesh = plsc.VectorSubcoreMesh(core_axis_name="c", subcore_axis_name="s")

    @functools.partial(
        pl.kernel,
        mesh=mesh,
        out_type=jax.ShapeDtypeStruct((_N_CORES, _N_PAD, EMB_C),
                                      jnp.float32),
        scratch_types=[
            pltpu.MemorySpace.VMEM_SHARED((_N_PAD, EMB_C), jnp.float32),
            pltpu.VMEM((_CHUNK, EMB_C), jnp.float32),
            pltpu.VMEM((_CHUNK,), jnp.int32),
        ],
    )
    def seg(tmp_hbm, src_hbm, zeros_hbm, out_hbm, t_sh, rows_v, idx_v):
        c = lax.axis_index("c")
        s = lax.axis_index("s")
        pltpu.sync_copy(zeros_hbm,
                        t_sh.at[pl.ds(s * _ROWS_PER_SUB, _ROWS_PER_SUB)])
        plsc.subcore_barrier()

        base0 = c * (E_C // _N_CORES) + s * _E_PER_SUB

        def body(j, carry):
            base = base0 + j * _CHUNK
            pltpu.sync_copy(tmp_hbm.at[pl.ds(base, _CHUNK)], rows_v)
            pltpu.sync_copy(src_hbm.at[pl.ds(base, _CHUNK)], idx_v)
            pltpu.sync_copy(rows_v, t_sh.at[idx_v], add=True)
            return carry

        lax.fori_loop(0, _N_CHUNKS, body, 0)
        plsc.subcore_barrier()
        pltpu.sync_copy(
            t_sh.at[pl.ds(s * _ROWS_PER_SUB, _ROWS_PER_SUB)],
            out_hbm.at[c, pl.ds(s * _ROWS_PER_SUB, _ROWS_PER_SUB)])

    return seg


_NODE_BLK = 200


def _chain_kernel(t0_ref, t1_ref, wup_ref, w1_ref, b1_ref, w2_ref, b2_ref,
                  w3_ref, b3_ref, wfin_ref, out_ref, acc_ref):
    i = pl.program_id(0)
    nsteps = pl.num_programs(0)

    @pl.when(i == 0)
    def _init():
        acc_ref[...] = jnp.zeros_like(acc_ref)

    t = t0_ref[...] + t1_ref[...]
    u = jnp.dot(t, wup_ref[...], preferred_element_type=jnp.float32)
    u = jnp.dot(u, w1_ref[...], preferred_element_type=jnp.float32) + b1_ref[...]
    u = jnp.dot(u, w2_ref[...], preferred_element_type=jnp.float32) + b2_ref[...]
    u = jnp.dot(u, w3_ref[...], preferred_element_type=jnp.float32) + b3_ref[...]
    y = jnp.dot(u, wfin_ref[...], preferred_element_type=jnp.float32)
    acc_ref[...] += jnp.sum(y, axis=0, keepdims=True)

    @pl.when(i == nsteps - 1)
    def _fin():
        out_ref[...] = acc_ref[...]


def _chain_stage(tparts, wup_t, w1_t, b1r, w2_t, b2r, w3_t, b3r, wfin_t):
    emb = tparts.shape[2]
    oe = wup_t.shape[1]
    nt = wfin_t.shape[1]
    full = lambda i: (0, 0)
    return pl.pallas_call(
        _chain_kernel,
        grid=(N_NODES_C // _NODE_BLK,),
        in_specs=[
            pl.BlockSpec((_NODE_BLK, emb), lambda i: (i, 0)),
            pl.BlockSpec((_NODE_BLK, emb), lambda i: (i, 0)),
            pl.BlockSpec((emb, oe), full),
            pl.BlockSpec((oe, oe), full),
            pl.BlockSpec((1, oe), full),
            pl.BlockSpec((oe, oe), full),
            pl.BlockSpec((1, oe), full),
            pl.BlockSpec((oe, oe), full),
            pl.BlockSpec((1, oe), full),
            pl.BlockSpec((oe, nt), full),
        ],
        out_specs=pl.BlockSpec((1, nt), full),
        out_shape=jax.ShapeDtypeStruct((1, nt), jnp.float32),
        scratch_shapes=[pltpu.VMEM((1, nt), jnp.float32)],
    )(tparts[0], tparts[1],
      wup_t, w1_t, b1r, w2_t, b2r, w3_t, b3r, wfin_t)


def kernel(m, rbf, edge_index, W_rbf, W_up, W1, b1, W2, b2, W3, b3, W_final):
    nr = rbf.shape[1]
    rbf_p = jnp.pad(rbf, ((0, 0), (0, RBF_PAD - nr)))
    wrbf_t = jnp.pad(W_rbf.T, ((0, RBF_PAD - nr), (0, 0)))
    src = edge_index[0]
    zeros = jnp.zeros((_ROWS_PER_SUB, EMB_C), jnp.float32)

    tmp = _edge_stage(rbf_p, m, wrbf_t)
    tparts = _make_seg_kernel()(tmp, src, zeros)
    return _chain_stage(tparts, W_up.T, W1.T, b1.reshape(1, -1),
                        W2.T, b2.reshape(1, -1), W3.T, b3.reshape(1, -1),
                        W_final.T)


# speedup vs baseline: 1.2974x; 1.2974x over previous
import functools

import jax
import jax.numpy as jnp
from jax import lax
from jax.experimental import pallas as pl
from jax.experimental.pallas import tpu as pltpu
from jax.experimental.pallas import tpu_sc as plsc

N_NODES_C = 10000
E_C = 320000
EMB_C = 128
RBF_PAD = 8


_EDGE_BLK = 8000


def _edge_kernel(rbf_ref, m_ref, wrbf_ref, tmp_ref):
    p = jnp.dot(rbf_ref[...], wrbf_ref[...],
                preferred_element_type=jnp.float32)
    tmp_ref[...] = m_ref[...] * p


def _edge_stage(rbf_p, m, wrbf_t):
    e = m.shape[0]
    return pl.pallas_call(
        _edge_kernel,
        grid=(e // _EDGE_BLK,),
        in_specs=[
            pl.BlockSpec((_EDGE_BLK, RBF_PAD), lambda i: (i, 0)),
            pl.BlockSpec((_EDGE_BLK, EMB_C), lambda i: (i, 0)),
            pl.BlockSpec((RBF_PAD, EMB_C), lambda i: (0, 0)),
        ],
        out_specs=pl.BlockSpec((_EDGE_BLK, EMB_C), lambda i: (i, 0)),
        out_shape=jax.ShapeDtypeStruct((e, EMB_C), jnp.float32),
    )(rbf_p, m, wrbf_t)


_CHUNK = 80
_N_CORES = 2
_N_SUB = 16
_E_PER_SUB = E_C // (_N_CORES * _N_SUB)
_N_CHUNKS = _E_PER_SUB // _CHUNK
_N_PAD = 10240
_ROWS_PER_SUB = _N_PAD // _N_SUB


def _make_seg_kernel():
    mesh = plsc.VectorSubcoreMesh(core_axis_name="c", subcore_axis_name="s")

    @functools.partial(
        pl.kernel,
        mesh=mesh,
        out_type=jax.ShapeDtypeStruct((_N_CORES, _N_PAD, EMB_C),
                                      jnp.float32),
        scratch_types=[
            pltpu.MemorySpace.VMEM_SHARED((_N_PAD, EMB_C), jnp.float32),
            pltpu.VMEM((_CHUNK, EMB_C), jnp.float32),
            pltpu.VMEM((_CHUNK, EMB_C), jnp.float32),
            pltpu.VMEM((_N_CHUNKS, 1, _CHUNK), jnp.int32),
            pltpu.SemaphoreType.DMA,
            pltpu.SemaphoreType.DMA,
        ],
    )
    def seg(tmp_hbm, src3_hbm, zeros_hbm, out_hbm,
            t_sh, rows0, rows1, idx_all, sem0, sem1):
        c = lax.axis_index("c")
        s = lax.axis_index("s")
        wid = c * _N_SUB + s
        pltpu.sync_copy(zeros_hbm,
                        t_sh.at[pl.ds(s * _ROWS_PER_SUB, _ROWS_PER_SUB)])
        pltpu.sync_copy(src3_hbm.at[pl.ds(wid * _N_CHUNKS, _N_CHUNKS)],
                        idx_all)
        plsc.subcore_barrier()

        base0 = wid * _E_PER_SUB
        bufs = (rows0, rows1)
        sems = (sem0, sem1)

        def load(b, j):
            return pltpu.make_async_copy(
                tmp_hbm.at[pl.ds(base0 + j * _CHUNK, _CHUNK)], bufs[b],
                sems[b])

        def scatter(b, j):
            pltpu.sync_copy(bufs[b], t_sh.at[idx_all.at[j, 0]], add=True)

        load(0, 0).start()
        load(1, 1).start()

        def body(p, carry):
            j0 = 2 * p
            load(0, j0).wait()
            scatter(0, j0)
            load(0, j0 + 2).start()

            j1 = j0 + 1
            load(1, j1).wait()
            scatter(1, j1)

            @pl.when(p < (_N_CHUNKS - 1) // 2 - 1)
            def _():
                load(1, j1 + 2).start()
            return carry

        lax.fori_loop(0, (_N_CHUNKS - 1) // 2, body, 0)
        load(0, _N_CHUNKS - 1).wait()
        scatter(0, _N_CHUNKS - 1)

        plsc.subcore_barrier()
        pltpu.sync_copy(
            t_sh.at[pl.ds(s * _ROWS_PER_SUB, _ROWS_PER_SUB)],
            out_hbm.at[c, pl.ds(s * _ROWS_PER_SUB, _ROWS_PER_SUB)])

    return seg


_NODE_BLK = 200


def _chain_kernel(t0_ref, t1_ref, wup_ref, w1_ref, b1_ref, w2_ref, b2_ref,
                  w3_ref, b3_ref, wfin_ref, out_ref, acc_ref):
    i = pl.program_id(0)
    nsteps = pl.num_programs(0)

    @pl.when(i == 0)
    def _init():
        acc_ref[...] = jnp.zeros_like(acc_ref)

    t = t0_ref[...] + t1_ref[...]
    u = jnp.dot(t, wup_ref[...], preferred_element_type=jnp.float32)
    u = jnp.dot(u, w1_ref[...], preferred_element_type=jnp.float32) + b1_ref[...]
    u = jnp.dot(u, w2_ref[...], preferred_element_type=jnp.float32) + b2_ref[...]
    u = jnp.dot(u, w3_ref[...], preferred_element_type=jnp.float32) + b3_ref[...]
    y = jnp.dot(u, wfin_ref[...], preferred_element_type=jnp.float32)
    acc_ref[...] += jnp.sum(y, axis=0, keepdims=True)

    @pl.when(i == nsteps - 1)
    def _fin():
        out_ref[...] = acc_ref[...]


def _chain_stage(tparts, wup_t, w1_t, b1r, w2_t, b2r, w3_t, b3r, wfin_t):
    emb = tparts.shape[2]
    oe = wup_t.shape[1]
    nt = wfin_t.shape[1]
    full = lambda i: (0, 0)
    return pl.pallas_call(
        _chain_kernel,
        grid=(N_NODES_C // _NODE_BLK,),
        in_specs=[
            pl.BlockSpec((_NODE_BLK, emb), lambda i: (i, 0)),
            pl.BlockSpec((_NODE_BLK, emb), lambda i: (i, 0)),
            pl.BlockSpec((emb, oe), full),
            pl.BlockSpec((oe, oe), full),
            pl.BlockSpec((1, oe), full),
            pl.BlockSpec((oe, oe), full),
            pl.BlockSpec((1, oe), full),
            pl.BlockSpec((oe, oe), full),
            pl.BlockSpec((1, oe), full),
            pl.BlockSpec((oe, nt), full),
        ],
        out_specs=pl.BlockSpec((1, nt), full),
        out_shape=jax.ShapeDtypeStruct((1, nt), jnp.float32),
        scratch_shapes=[pltpu.VMEM((1, nt), jnp.float32)],
    )(tparts[0], tparts[1],
      wup_t, w1_t, b1r, w2_t, b2r, w3_t, b3r, wfin_t)


def kernel(m, rbf, edge_index, W_rbf, W_up, W1, b1, W2, b2, W3, b3, W_final):
    nr = rbf.shape[1]
    rbf_p = jnp.pad(rbf, ((0, 0), (0, RBF_PAD - nr)))
    wrbf_t = jnp.pad(W_rbf.T, ((0, RBF_PAD - nr), (0, 0)))
    src3 = edge_index[0].reshape(E_C // _CHUNK, 1, _CHUNK)
    zeros = jnp.zeros((_ROWS_PER_SUB, EMB_C), jnp.float32)

    tmp = _edge_stage(rbf_p, m, wrbf_t)
    tparts = _make_seg_kernel()(tmp, src3, zeros)
    return _chain_stage(tparts, W_up.T, W1.T, b1.reshape(1, -1),
                        W2.T, b2.reshape(1, -1), W3.T, b3.reshape(1, -1),
                        W_final.T)


# speedup vs baseline: 1.3625x; 1.0502x over previous
import functools

import jax
import jax.numpy as jnp
from jax import lax
from jax.experimental import pallas as pl
from jax.experimental.pallas import tpu as pltpu
from jax.experimental.pallas import tpu_sc as plsc

N_NODES_C = 10000
E_C = 320000
EMB_C = 128
RBF_PAD = 8


_EDGE_BLK = 8000


def _edge_kernel(rbf_ref, m_ref, wrbf_ref, tmp_ref):
    p = jnp.dot(rbf_ref[...], wrbf_ref[...],
                preferred_element_type=jnp.float32)
    tmp_ref[...] = m_ref[...] * p


def _edge_stage(rbf_p, m, wrbf_t):
    e = m.shape[0]
    return pl.pallas_call(
        _edge_kernel,
        grid=(e // _EDGE_BLK,),
        in_specs=[
            pl.BlockSpec((_EDGE_BLK, RBF_PAD), lambda i: (i, 0)),
            pl.BlockSpec((_EDGE_BLK, EMB_C), lambda i: (i, 0)),
            pl.BlockSpec((RBF_PAD, EMB_C), lambda i: (0, 0)),
        ],
        out_specs=pl.BlockSpec((_EDGE_BLK, EMB_C), lambda i: (i, 0)),
        out_shape=jax.ShapeDtypeStruct((e, EMB_C), jnp.float32),
    )(rbf_p, m, wrbf_t)


_CHUNK = 80
_N_CORES = 2
_N_SUB = 16
_E_PER_SUB = E_C // (_N_CORES * _N_SUB)
_N_CHUNKS = _E_PER_SUB // _CHUNK
_N_PAD = 10240
_ROWS_PER_SUB = _N_PAD // _N_SUB


def _make_seg_kernel():
    mesh = plsc.VectorSubcoreMesh(core_axis_name="c", subcore_axis_name="s")

    @functools.partial(
        pl.kernel,
        mesh=mesh,
        out_type=jax.ShapeDtypeStruct((_N_CORES, _N_PAD, EMB_C),
                                      jnp.float32),
        scratch_types=[
            pltpu.MemorySpace.VMEM_SHARED((_N_PAD, EMB_C), jnp.float32),
            pltpu.VMEM((_CHUNK, EMB_C), jnp.float32),
            pltpu.VMEM((_CHUNK, EMB_C), jnp.float32),
            pltpu.VMEM((_CHUNK, EMB_C), jnp.float32),
            pltpu.VMEM((_N_CHUNKS, 1, _CHUNK), jnp.int32),
            pltpu.SemaphoreType.DMA,
            pltpu.SemaphoreType.DMA,
            pltpu.SemaphoreType.DMA,
        ],
    )
    def seg(tmp_hbm, src3_hbm, zeros_hbm, out_hbm,
            t_sh, rows0, rows1, rows2, idx_all,
            sem0, sem1, sem2):
        c = lax.axis_index("c")
        s = lax.axis_index("s")
        wid = c * _N_SUB + s
        pltpu.sync_copy(zeros_hbm,
                        t_sh.at[pl.ds(s * _ROWS_PER_SUB, _ROWS_PER_SUB)])
        pltpu.sync_copy(src3_hbm.at[pl.ds(wid * _N_CHUNKS, _N_CHUNKS)],
                        idx_all)
        plsc.subcore_barrier()

        base0 = wid * _E_PER_SUB
        bufs = (rows0, rows1, rows2)
        sems = (sem0, sem1, sem2)
        nbuf = 3

        def load(b, j):
            return pltpu.make_async_copy(
                tmp_hbm.at[pl.ds(base0 + j * _CHUNK, _CHUNK)], bufs[b],
                sems[b])

        def scatter(b, j):
            pltpu.sync_copy(bufs[b], t_sh.at[idx_all.at[j, 0]], add=True)

        for b in range(nbuf):
            load(b, b).start()

        def body(p, carry):
            for k in range(nbuf):
                j = nbuf * p + k
                load(k, j).wait()
                scatter(k, j)
                nxt = j + nbuf

                @pl.when(nxt < _N_CHUNKS)
                def _():
                    load(k, nxt).start()
            return carry

        n_full = (_N_CHUNKS // nbuf) * nbuf
        lax.fori_loop(0, _N_CHUNKS // nbuf, body, 0)
        for k in range(_N_CHUNKS - n_full):
            load(k, n_full + k).wait()
            scatter(k, n_full + k)

        plsc.subcore_barrier()
        pltpu.sync_copy(
            t_sh.at[pl.ds(s * _ROWS_PER_SUB, _ROWS_PER_SUB)],
            out_hbm.at[c, pl.ds(s * _ROWS_PER_SUB, _ROWS_PER_SUB)])

    return seg


_NODE_BLK = 200


def _chain_kernel(t0_ref, t1_ref, wup_ref, w1_ref, b1_ref, w2_ref, b2_ref,
                  w3_ref, b3_ref, wfin_ref, out_ref, acc_ref):
    i = pl.program_id(0)
    nsteps = pl.num_programs(0)

    @pl.when(i == 0)
    def _init():
        acc_ref[...] = jnp.zeros_like(acc_ref)

    t = t0_ref[...] + t1_ref[...]
    u = jnp.dot(t, wup_ref[...], preferred_element_type=jnp.float32)
    u = jnp.dot(u, w1_ref[...], preferred_element_type=jnp.float32) + b1_ref[...]
    u = jnp.dot(u, w2_ref[...], preferred_element_type=jnp.float32) + b2_ref[...]
    u = jnp.dot(u, w3_ref[...], preferred_element_type=jnp.float32) + b3_ref[...]
    y = jnp.dot(u, wfin_ref[...], preferred_element_type=jnp.float32)
    acc_ref[...] += jnp.sum(y, axis=0, keepdims=True)

    @pl.when(i == nsteps - 1)
    def _fin():
        out_ref[...] = acc_ref[...]


def _chain_stage(tparts, wup_t, w1_t, b1r, w2_t, b2r, w3_t, b3r, wfin_t):
    emb = tparts.shape[2]
    oe = wup_t.shape[1]
    nt = wfin_t.shape[1]
    full = lambda i: (0, 0)
    return pl.pallas_call(
        _chain_kernel,
        grid=(N_NODES_C // _NODE_BLK,),
        in_specs=[
            pl.BlockSpec((_NODE_BLK, emb), lambda i: (i, 0)),
            pl.BlockSpec((_NODE_BLK, emb), lambda i: (i, 0)),
            pl.BlockSpec((emb, oe), full),
            pl.BlockSpec((oe, oe), full),
            pl.BlockSpec((1, oe), full),
            pl.BlockSpec((oe, oe), full),
            pl.BlockSpec((1, oe), full),
            pl.BlockSpec((oe, oe), full),
            pl.BlockSpec((1, oe), full),
            pl.BlockSpec((oe, nt), full),
        ],
        out_specs=pl.BlockSpec((1, nt), full),
        out_shape=jax.ShapeDtypeStruct((1, nt), jnp.float32),
        scratch_shapes=[pltpu.VMEM((1, nt), jnp.float32)],
    )(tparts[0], tparts[1],
      wup_t, w1_t, b1r, w2_t, b2r, w3_t, b3r, wfin_t)


def kernel(m, rbf, edge_index, W_rbf, W_up, W1, b1, W2, b2, W3, b3, W_final):
    nr = rbf.shape[1]
    rbf_p = jnp.pad(rbf, ((0, 0), (0, RBF_PAD - nr)))
    wrbf_t = jnp.pad(W_rbf.T, ((0, RBF_PAD - nr), (0, 0)))
    src3 = edge_index[0].reshape(E_C // _CHUNK, 1, _CHUNK)
    zeros = jnp.zeros((_ROWS_PER_SUB, EMB_C), jnp.float32)

    tmp = _edge_stage(rbf_p, m, wrbf_t)
    tparts = _make_seg_kernel()(tmp, src3, zeros)
    return _chain_stage(tparts, W_up.T, W1.T, b1.reshape(1, -1),
                        W2.T, b2.reshape(1, -1), W3.T, b3.reshape(1, -1),
                        W_final.T)
